# baseline (speedup 1.0000x reference)
import jax
import jax.numpy as jnp
from jax.experimental import pallas as pl
from jax.experimental.pallas import tpu as pltpu

_LANES = 128
_NCLS = 29


def _fused_body(x_ref, w_ref, b_ref, out_ref, acc_ref):
    i = pl.program_id(0)
    n = pl.num_programs(0)

    @pl.when(i == 0)
    def _init():
        acc_ref[...] = jnp.zeros_like(acc_ref)

    logits = jnp.dot(
        x_ref[...], w_ref[...],
        preferred_element_type=jnp.float32,
        precision=jax.lax.Precision.HIGHEST,
    ) + b_ref[...]
    lane = jax.lax.broadcasted_iota(jnp.int32, logits.shape, 1)
    rowmax = jnp.max(logits, axis=1, keepdims=True)
    vote = jnp.min(jnp.where(logits == rowmax, lane, _LANES), axis=1,
                   keepdims=True)
    onehot = (lane == vote).astype(jnp.float32)
    acc_ref[...] += jnp.sum(onehot, axis=0, keepdims=True)

    @pl.when(i == n - 1)
    def _fin():
        counts = acc_ref[...]
        cmax = jnp.max(counts)
        l1 = jax.lax.broadcasted_iota(jnp.int32, counts.shape, 1)
        winner = jnp.min(jnp.where(counts == cmax, l1, _LANES))
        out_ref[...] = (l1 == winner).astype(jnp.float32)


def kernel(x, W, b):
    _, T, C = x.shape
    xs = x.reshape(T, C)
    Wp = jnp.zeros((C, _LANES), jnp.float32).at[:, :_NCLS].set(W.T)
    bp = jnp.full((1, _LANES), -1e30, jnp.float32).at[0, :_NCLS].set(b)
    Tb = 1024
    out = pl.pallas_call(
        _fused_body,
        grid=(T // Tb,),
        in_specs=[
            pl.BlockSpec((Tb, C), lambda i: (i, 0)),
            pl.BlockSpec((C, _LANES), lambda i: (0, 0)),
            pl.BlockSpec((1, _LANES), lambda i: (0, 0)),
        ],
        out_specs=pl.BlockSpec((1, _LANES), lambda i: (0, 0)),
        out_shape=jax.ShapeDtypeStruct((1, _LANES), jnp.float32),
        scratch_shapes=[pltpu.VMEM((1, _LANES), jnp.float32)],
    )(xs, Wp, bp)
    return out[:, :_NCLS]


# speedup vs baseline: 2.1455x; 2.1455x over previous
import jax
import jax.numpy as jnp
from jax.experimental import pallas as pl
from jax.experimental.pallas import tpu as pltpu

_LANES = 128
_NCLS = 29


def _fused_body(x_ref, w_ref, b_ref, out_ref, acc_ref):
    i = pl.program_id(0)
    n = pl.num_programs(0)

    @pl.when(i == 0)
    def _init():
        acc_ref[...] = jnp.zeros_like(acc_ref)

    logits = jnp.dot(
        x_ref[...], w_ref[...],
        preferred_element_type=jnp.float32,
    ) + b_ref[...]
    lane = jax.lax.broadcasted_iota(jnp.int32, logits.shape, 1)
    rowmax = jnp.max(logits, axis=1, keepdims=True)
    vote = jnp.min(jnp.where(logits == rowmax, lane, _LANES), axis=1,
                   keepdims=True)
    onehot = (lane == vote).astype(jnp.float32)
    acc_ref[...] += jnp.sum(onehot, axis=0, keepdims=True)

    @pl.when(i == n - 1)
    def _fin():
        counts = acc_ref[...]
        cmax = jnp.max(counts)
        l1 = jax.lax.broadcasted_iota(jnp.int32, counts.shape, 1)
        winner = jnp.min(jnp.where(counts == cmax, l1, _LANES))
        out_ref[...] = (l1 == winner).astype(jnp.float32)


def kernel(x, W, b):
    _, T, C = x.shape
    xs = x.reshape(T, C)
    Wp = jnp.zeros((C, _LANES), jnp.float32).at[:, :_NCLS].set(W.T)
    bp = jnp.full((1, _LANES), -1e30, jnp.float32).at[0, :_NCLS].set(b)
    Tb = 1024
    out = pl.pallas_call(
        _fused_body,
        grid=(T // Tb,),
        in_specs=[
            pl.BlockSpec((Tb, C), lambda i: (i, 0)),
            pl.BlockSpec((C, _LANES), lambda i: (0, 0)),
            pl.BlockSpec((1, _LANES), lambda i: (0, 0)),
        ],
        out_specs=pl.BlockSpec((1, _LANES), lambda i: (0, 0)),
        out_shape=jax.ShapeDtypeStruct((1, _LANES), jnp.float32),
        scratch_shapes=[pltpu.VMEM((1, _LANES), jnp.float32)],
    )(xs, Wp, bp)
    return out[:, :_NCLS]


# speedup vs baseline: 2.5221x; 1.1755x over previous
import jax
import jax.numpy as jnp
from jax.experimental import pallas as pl
from jax.experimental.pallas import tpu as pltpu

_LANES = 128
_NCLS = 29


def _fused_body(x_ref, w_ref, b_ref, out_ref, acc_ref):
    i = pl.program_id(0)
    n = pl.num_programs(0)

    @pl.when(i == 0)
    def _init():
        acc_ref[...] = jnp.zeros_like(acc_ref)

    logits = jnp.dot(
        x_ref[...], w_ref[...],
        preferred_element_type=jnp.float32,
    ) + b_ref[...]
    lane = jax.lax.broadcasted_iota(jnp.int32, logits.shape, 1)
    rowmax = jnp.max(logits, axis=1, keepdims=True)
    vote = jnp.min(jnp.where(logits == rowmax, lane, _LANES), axis=1,
                   keepdims=True)
    onehot = (lane == vote).astype(jnp.float32)
    acc_ref[...] += jnp.sum(onehot, axis=0, keepdims=True)

    @pl.when(i == n - 1)
    def _fin():
        counts = acc_ref[...]
        cmax = jnp.max(counts)
        l1 = jax.lax.broadcasted_iota(jnp.int32, counts.shape, 1)
        winner = jnp.min(jnp.where(counts == cmax, l1, _LANES))
        out_ref[...] = (l1 == winner).astype(jnp.float32)


def kernel(x, W, b):
    _, T, C = x.shape
    xs = x.reshape(T, C)
    Wp = jnp.zeros((C, _LANES), jnp.float32).at[:, :_NCLS].set(W.T)
    bp = jnp.full((1, _LANES), -1e30, jnp.float32).at[0, :_NCLS].set(b)
    Tb = 2048
    out = pl.pallas_call(
        _fused_body,
        grid=(T // Tb,),
        in_specs=[
            pl.BlockSpec((Tb, C), lambda i: (i, 0)),
            pl.BlockSpec((C, _LANES), lambda i: (0, 0)),
            pl.BlockSpec((1, _LANES), lambda i: (0, 0)),
        ],
        out_specs=pl.BlockSpec((1, _LANES), lambda i: (0, 0)),
        out_shape=jax.ShapeDtypeStruct((1, _LANES), jnp.float32),
        scratch_shapes=[pltpu.VMEM((1, _LANES), jnp.float32)],
    )(xs, Wp, bp)
    return out[:, :_NCLS]
